# baseline (speedup 1.0000x reference)
import functools

import jax
import jax.numpy as jnp
from jax import lax
from jax.experimental import pallas as pl
from jax.experimental.pallas import tpu as pltpu
from jax.experimental.pallas import tpu_sc as plsc

_N = 10000
_E = 320000
_D = 128
_RB = 128
_G = 79
_NP = _G * _RB

_NT = 32
_CH = 128
_NCH = 80
_EPT = _CH * _NCH
_EP = _NT * _EPT
_SLICE = _NP // 16
_NCH2 = 2 * _NCH
_NC0 = 44


def _tc_a_body(p0_ref, p1_ref, x_ref, w1_ref, dis_ref, z1_ref):
    deg = p0_ref[...] + p1_ref[...] + 1.0
    dis = jax.lax.rsqrt(deg)
    dis_ref[...] = dis
    z1_ref[...] = dis * jax.lax.dot_general(
        x_ref[...], w1_ref[...], (((1,), (0,)), ((), ())),
        preferred_element_type=jnp.float32)


def _tc_a(p0, p1, x, w1):
    col = pl.BlockSpec((_RB, 1), lambda i: (i, 0))
    mat = pl.BlockSpec((_RB, _D), lambda i: (i, 0))
    wsp = pl.BlockSpec((_D, _D), lambda i: (0, 0))
    return pl.pallas_call(
        _tc_a_body,
        grid=(_G,),
        in_specs=[col, col, mat, wsp],
        out_specs=[col, mat],
        out_shape=[jax.ShapeDtypeStruct((_NP, 1), jnp.float32),
                   jax.ShapeDtypeStruct((_NP, _D), jnp.float32)],
    )(p0, p1, x, w1)


def _tc_b_body(q0_ref, q1_ref, z_ref, dis_ref, b_ref, s_ref, t_ref, w2_ref,
               h_ref, z2_ref):
    dis = dis_ref[...]
    gcn = dis * (q0_ref[...] + q1_ref[...] + z_ref[...]) + b_ref[...]
    h = jnp.maximum(gcn, 0.0) * s_ref[...] + t_ref[...]
    h_ref[...] = h
    z2_ref[...] = dis * jax.lax.dot_general(
        h, w2_ref[...], (((1,), (0,)), ((), ())),
        preferred_element_type=jnp.float32)


def _tc_b(q0, q1, z, dis, b, s, t, w2):
    col = pl.BlockSpec((_RB, 1), lambda i: (i, 0))
    mat = pl.BlockSpec((_RB, _D), lambda i: (i, 0))
    row = pl.BlockSpec((1, _D), lambda i: (0, 0))
    wsp = pl.BlockSpec((_D, _D), lambda i: (0, 0))
    return pl.pallas_call(
        _tc_b_body,
        grid=(_G,),
        in_specs=[mat, mat, mat, col, row, row, row, wsp],
        out_specs=[mat, mat],
        out_shape=[jax.ShapeDtypeStruct((_NP, _D), jnp.float32),
                   jax.ShapeDtypeStruct((_NP, _D), jnp.float32)],
    )(q0, q1, z, dis, b, s, t, w2)


def _tc_c_body(q0_ref, q1_ref, z2_ref, dis_ref, b_ref, s_ref, t_ref,
               h1_ref, x_ref, w1a_ref, w1b_ref, bias1_ref, w2t_ref, bias2_ref,
               wab_ref, wc_ref, linb_ref, out_ref):
    dis = dis_ref[...]
    gcn = dis * (q0_ref[...] + q1_ref[...] + z2_ref[...]) + b_ref[...]
    h2 = jnp.maximum(gcn, 0.0) * s_ref[...] + t_ref[...]
    h1 = h1_ref[...]
    g1 = (jax.lax.dot_general(h1, w1a_ref[...], (((1,), (0,)), ((), ())),
                              preferred_element_type=jnp.float32)
          + jax.lax.dot_general(h2, w1b_ref[...], (((1,), (0,)), ((), ())),
                                preferred_element_type=jnp.float32)
          + bias1_ref[...])
    i1 = jax.nn.sigmoid(g1[:, :_D])
    gg1 = jnp.tanh(g1[:, 2 * _D:3 * _D])
    o1 = jax.nn.sigmoid(g1[:, 3 * _D:])
    r1 = o1 * jnp.tanh(i1 * gg1)
    g2 = jax.lax.dot_general(r1, w2t_ref[...], (((1,), (0,)), ((), ())),
                             preferred_element_type=jnp.float32) + bias2_ref[...]
    i2 = jax.nn.sigmoid(g2[:, :_D])
    gg2 = jnp.tanh(g2[:, 2 * _D:3 * _D])
    o2 = jax.nn.sigmoid(g2[:, 3 * _D:])
    r2 = o2 * jnp.tanh(i2 * gg2)
    acc = (jax.lax.dot_general(jnp.maximum(r2, 0.0), wab_ref[...],
                               (((1,), (0,)), ((), ())),
                               preferred_element_type=jnp.float32)
           + jax.lax.dot_general(jnp.maximum(x_ref[...], 0.0), wc_ref[...],
                                 (((1,), (0,)), ((), ())),
                                 preferred_element_type=jnp.float32))
    out_ref[...] = jnp.tanh(acc + linb_ref[...])


def _tc_c(q0, q1, z2, dis, b, s, t, h1, x, w1a, w1b, bias1, w2t, bias2,
          wab, wc, linb):
    col = pl.BlockSpec((_RB, 1), lambda i: (i, 0))
    mat = pl.BlockSpec((_RB, _D), lambda i: (i, 0))
    row = pl.BlockSpec((1, _D), lambda i: (0, 0))
    w4 = pl.BlockSpec((_D, 4 * _D), lambda i: (0, 0))
    row4 = pl.BlockSpec((1, 4 * _D), lambda i: (0, 0))
    wv = pl.BlockSpec((_D, 1), lambda i: (0, 0))
    sc = pl.BlockSpec((1, 1), lambda i: (0, 0))
    return pl.pallas_call(
        _tc_c_body,
        grid=(_G,),
        in_specs=[mat, mat, mat, col, row, row, row, mat, mat,
                  w4, w4, row4, w4, row4, wv, wv, sc],
        out_specs=col,
        out_shape=jax.ShapeDtypeStruct((_NP, 1), jnp.float32),
    )(q0, q1, z2, dis, b, s, t, h1, x, w1a, w1b, bias1, w2t, bias2,
      wab, wc, linb)


_MESH = plsc.VectorSubcoreMesh(core_axis_name="c", subcore_axis_name="s")


@functools.partial(
    pl.kernel,
    mesh=_MESH,
    out_type=jax.ShapeDtypeStruct((2 * _NP,), jnp.float32),
    scratch_types=[
        pltpu.VMEM((_CH,), jnp.int32),
        pltpu.VMEM((_CH,), jnp.int32),
        pltpu.VMEM((_CH,), jnp.float32),
        pltpu.VMEM((_CH,), jnp.float32),
        pltpu.VMEM((_SLICE,), jnp.float32),
        pltpu.VMEM_SHARED((_NP,), jnp.float32),
        pltpu.SemaphoreType.DMA,
        pltpu.SemaphoreType.DMA,
    ],
)
def _sc_deg(col_hbm, w_hbm, out_hbm, cidx0, cidx1, wch0, wch1, dbuf, acc,
            ds0, ds1):
    cid = lax.axis_index("c")
    sid = lax.axis_index("s")
    wid = cid * 16 + sid
    base = wid * _EPT

    def zero16(i, carry):
        dbuf[pl.ds(i * 16, 16)] = jnp.zeros((16,), jnp.float32)
        return carry

    lax.fori_loop(0, _SLICE // 16, zero16, 0)
    dbuf[pl.ds(_SLICE - 16, 16)] = jnp.zeros((16,), jnp.float32)
    pltpu.sync_copy(dbuf, acc.at[pl.ds(sid * _SLICE, _SLICE)])
    plsc.subcore_barrier()

    pltpu.async_copy(col_hbm.at[pl.ds(base, _CH)], cidx0, ds0)
    pltpu.async_copy(w_hbm.at[pl.ds(base, _CH)], wch0, ds0)

    def pair(p, carry):
        for b, cidx, wch, sem, ocidx, owch, osem in (
                (0, cidx0, wch0, ds0, cidx1, wch1, ds1),
                (1, cidx1, wch1, ds1, cidx0, wch0, ds0)):
            k = 2 * p + b
            pltpu.make_async_copy(
                col_hbm.at[pl.ds(base + k * _CH, _CH)], cidx, sem).wait()
            pltpu.make_async_copy(
                w_hbm.at[pl.ds(base + k * _CH, _CH)], wch, sem).wait()

            @pl.when(k + 1 < _NCH)
            def _():
                pltpu.async_copy(
                    col_hbm.at[pl.ds(base + (k + 1) * _CH, _CH)], ocidx, osem)
                pltpu.async_copy(
                    w_hbm.at[pl.ds(base + (k + 1) * _CH, _CH)], owch, osem)

            pltpu.sync_copy(wch, acc.at[cidx], add=True)
        return carry

    lax.fori_loop(0, _NCH // 2, pair, 0)
    plsc.subcore_barrier()
    pltpu.sync_copy(acc.at[pl.ds(sid * _SLICE, _SLICE)], dbuf)
    pltpu.sync_copy(dbuf, out_hbm.at[pl.ds(cid * _NP + sid * _SLICE, _SLICE)])


@functools.partial(
    pl.kernel,
    mesh=_MESH,
    out_type=jax.ShapeDtypeStruct((2, _NP, _D), jnp.float32),
    scratch_types=[
        pltpu.VMEM((_CH,), jnp.int32),
        pltpu.VMEM((_CH,), jnp.int32),
        pltpu.VMEM((_CH,), jnp.int32),
        pltpu.VMEM((_CH,), jnp.int32),
        pltpu.VMEM((_CH,), jnp.float32),
        pltpu.VMEM((_CH,), jnp.float32),
        pltpu.VMEM((_CH, _D), jnp.float32),
        pltpu.VMEM((_CH, _D), jnp.float32),
        pltpu.VMEM_SHARED((_NP, _D), jnp.float32),
        pltpu.SemaphoreType.DMA,
        pltpu.SemaphoreType.DMA,
        pltpu.SemaphoreType.DMA,
        pltpu.SemaphoreType.DMA,
    ],
)
def _sc_agg(z_hbm, row_hbm, col_hbm, w_hbm, out_hbm,
            ridx0, ridx1, cidx0, cidx1, wch0, wch1, rows0, rows1, acc,
            is0, is1, gs0, gs1):
    cid = lax.axis_index("c")
    sid = lax.axis_index("s")
    ncht = jnp.where(cid == 0, _NC0, _NCH2 - _NC0)
    base = (jnp.where(cid == 0, sid * _NC0, 16 * _NC0 + sid * (_NCH2 - _NC0))
            * _CH)

    def zrow(r, carry):
        for j in range(8):
            rows0[r, pl.ds(j * 16, 16)] = jnp.zeros((16,), jnp.float32)
        return carry

    lax.fori_loop(0, _CH, zrow, 0)
    for p, sz in enumerate((_CH, _CH, _CH, _CH, _SLICE - 4 * _CH)):
        pltpu.sync_copy(rows0.at[pl.ds(0, sz)],
                        acc.at[pl.ds(sid * _SLICE + p * _CH, sz)])
    plsc.subcore_barrier()

    def idx_issue(off, ridx, cidx, wch, isem):
        pltpu.async_copy(row_hbm.at[pl.ds(off, _CH)], ridx, isem)
        pltpu.async_copy(col_hbm.at[pl.ds(off, _CH)], cidx, isem)
        pltpu.async_copy(w_hbm.at[pl.ds(off, _CH)], wch, isem)

    def idx_wait(off, ridx, cidx, wch, isem):
        pltpu.make_async_copy(row_hbm.at[pl.ds(off, _CH)], ridx, isem).wait()
        pltpu.make_async_copy(col_hbm.at[pl.ds(off, _CH)], cidx, isem).wait()
        pltpu.make_async_copy(w_hbm.at[pl.ds(off, _CH)], wch, isem).wait()

    idx_issue(base, ridx0, cidx0, wch0, is0)
    idx_wait(base, ridx0, cidx0, wch0, is0)
    pltpu.async_copy(z_hbm.at[ridx0], rows0, gs0)

    def pair(p, carry):
        for b, ridx, cidx, wch, isem, rows, gs, \
                oridx, ocidx, owch, oisem, orows, ogs in (
                (0, ridx0, cidx0, wch0, is0, rows0, gs0,
                 ridx1, cidx1, wch1, is1, rows1, gs1),
                (1, ridx1, cidx1, wch1, is1, rows1, gs1,
                 ridx0, cidx0, wch0, is0, rows0, gs0)):
            k = 2 * p + b
            nxt_off = base + (k + 1) * _CH

            @pl.when(k + 1 < ncht)
            def _():
                idx_issue(nxt_off, oridx, ocidx, owch, oisem)

            pltpu.make_async_copy(z_hbm.at[ridx], rows, gs).wait()

            def scale(g, c2):
                wv16 = wch[pl.ds(g * 16, 16)]
                for l in range(16):
                    e = g * 16 + l
                    wv = jnp.full((16,), wv16[l])
                    for j in range(8):
                        sl = pl.ds(j * 16, 16)
                        rows[e, sl] = rows[e, sl] * wv
                return c2

            lax.fori_loop(0, _CH // 16, scale, 0)

            @pl.when(k + 1 < ncht)
            def _():
                idx_wait(nxt_off, oridx, ocidx, owch, oisem)
                pltpu.async_copy(z_hbm.at[oridx], orows, ogs)

            pltpu.sync_copy(rows, acc.at[cidx], add=True)
        return carry

    lax.fori_loop(0, ncht // 2, pair, 0)
    plsc.subcore_barrier()
    for p, sz in enumerate((_CH, _CH, _CH, _CH, _SLICE - 4 * _CH)):
        off = sid * _SLICE + p * _CH
        pltpu.sync_copy(acc.at[pl.ds(off, sz)], rows0.at[pl.ds(0, sz)])
        pltpu.sync_copy(rows0.at[pl.ds(0, sz)], out_hbm.at[cid, pl.ds(off, sz)])


def kernel(x, edge_index, edge_weight, W1, b1, bn1_g, bn1_b, bn1_rm, bn1_rv,
           W2, b2, bn2_g, bn2_b, bn2_rm, bn2_rv,
           l1_wih, l1_whh, l1_bih, l1_bhh, l2_wih, l2_whh, l2_bih, l2_bhh,
           lin_w, lin_b):
    f32 = jnp.float32
    row = edge_index[0]
    col = edge_index[1]
    xp = jnp.zeros((_NP, _D), f32).at[:_N].set(x)
    rowp = jnp.zeros((_EP,), jnp.int32).at[:_E].set(row)
    colp = jnp.zeros((_EP,), jnp.int32).at[:_E].set(col)
    wp = jnp.zeros((_EP,), f32).at[:_E].set(edge_weight)

    s1 = (bn1_g / jnp.sqrt(bn1_rv + 1e-5)).reshape(1, _D)
    t1 = (bn1_b - bn1_rm * s1[0]).reshape(1, _D)
    s2 = (bn2_g / jnp.sqrt(bn2_rv + 1e-5)).reshape(1, _D)
    t2 = (bn2_b - bn2_rm * s2[0]).reshape(1, _D)

    w1t = l1_wih.T
    w1a = w1t[:_D]
    w1b = w1t[_D:]
    bias1 = (l1_bih + l1_bhh).reshape(1, 4 * _D)
    w2t = l2_wih.T
    bias2 = (l2_bih + l2_bhh).reshape(1, 4 * _D)
    wab = (lin_w[0, :_D] + lin_w[0, _D:2 * _D]).reshape(_D, 1)
    wc = lin_w[0, 2 * _D:].reshape(_D, 1)
    linb = lin_b.reshape(1, 1)

    deg_p = _sc_deg(colp, wp)
    dis, z1 = _tc_a(deg_p[:_NP].reshape(_NP, 1), deg_p[_NP:].reshape(_NP, 1),
                    xp, W1)

    q1 = _sc_agg(z1, rowp, colp, wp)
    h1, z2 = _tc_b(q1[0], q1[1], z1, dis, b1.reshape(1, _D), s1, t1, W2)

    q2 = _sc_agg(z2, rowp, colp, wp)
    out = _tc_c(q2[0], q2[1], z2, dis, b2.reshape(1, _D), s2, t2, h1, xp,
                w1a, w1b, bias1, w2t, bias2, wab, wc, linb)
    return out[:_N]


# speedup vs baseline: 1.2345x; 1.2345x over previous
import functools

import jax
import jax.numpy as jnp
from jax import lax
from jax.experimental import pallas as pl
from jax.experimental.pallas import tpu as pltpu
from jax.experimental.pallas import tpu_sc as plsc

_N = 10000
_E = 320000
_D = 128
_RB = 128
_G = 79
_NP = _G * _RB

_NT = 32
_CH = 128
_NCH = 80
_EPT = _CH * _NCH
_EP = _NT * _EPT
_SLICE = _NP // 16
_NCH2 = 2 * _NCH
_NC0 = 116


def _tc_a_body(p0_ref, p1_ref, x_ref, w1_ref, dis_ref, z1_ref):
    deg = p0_ref[...] + p1_ref[...] + 1.0
    dis = jax.lax.rsqrt(deg)
    dis_ref[...] = dis
    z1_ref[...] = dis * jax.lax.dot_general(
        x_ref[...], w1_ref[...], (((1,), (0,)), ((), ())),
        preferred_element_type=jnp.float32)


def _tc_a(p0, p1, x, w1):
    col = pl.BlockSpec((_RB, 1), lambda i: (i, 0))
    mat = pl.BlockSpec((_RB, _D), lambda i: (i, 0))
    wsp = pl.BlockSpec((_D, _D), lambda i: (0, 0))
    return pl.pallas_call(
        _tc_a_body,
        grid=(_G,),
        in_specs=[col, col, mat, wsp],
        out_specs=[col, mat],
        out_shape=[jax.ShapeDtypeStruct((_NP, 1), jnp.float32),
                   jax.ShapeDtypeStruct((_NP, _D), jnp.float32)],
    )(p0, p1, x, w1)


def _tc_b_body(q0_ref, q1_ref, z_ref, dis_ref, b_ref, s_ref, t_ref, w2_ref,
               h_ref, z2_ref):
    dis = dis_ref[...]
    gcn = dis * (q0_ref[...] + q1_ref[...] + z_ref[...]) + b_ref[...]
    h = jnp.maximum(gcn, 0.0) * s_ref[...] + t_ref[...]
    h_ref[...] = h
    z2_ref[...] = dis * jax.lax.dot_general(
        h, w2_ref[...], (((1,), (0,)), ((), ())),
        preferred_element_type=jnp.float32)


def _tc_b(q0, q1, z, dis, b, s, t, w2):
    col = pl.BlockSpec((_RB, 1), lambda i: (i, 0))
    mat = pl.BlockSpec((_RB, _D), lambda i: (i, 0))
    row = pl.BlockSpec((1, _D), lambda i: (0, 0))
    wsp = pl.BlockSpec((_D, _D), lambda i: (0, 0))
    return pl.pallas_call(
        _tc_b_body,
        grid=(_G,),
        in_specs=[mat, mat, mat, col, row, row, row, wsp],
        out_specs=[mat, mat],
        out_shape=[jax.ShapeDtypeStruct((_NP, _D), jnp.float32),
                   jax.ShapeDtypeStruct((_NP, _D), jnp.float32)],
    )(q0, q1, z, dis, b, s, t, w2)


def _tc_c_body(q0_ref, q1_ref, z2_ref, dis_ref, b_ref, s_ref, t_ref,
               h1_ref, x_ref, w1a_ref, w1b_ref, bias1_ref, w2t_ref, bias2_ref,
               wab_ref, wc_ref, linb_ref, out_ref):
    dis = dis_ref[...]
    gcn = dis * (q0_ref[...] + q1_ref[...] + z2_ref[...]) + b_ref[...]
    h2 = jnp.maximum(gcn, 0.0) * s_ref[...] + t_ref[...]
    h1 = h1_ref[...]
    g1 = (jax.lax.dot_general(h1, w1a_ref[...], (((1,), (0,)), ((), ())),
                              preferred_element_type=jnp.float32)
          + jax.lax.dot_general(h2, w1b_ref[...], (((1,), (0,)), ((), ())),
                                preferred_element_type=jnp.float32)
          + bias1_ref[...])
    i1 = jax.nn.sigmoid(g1[:, :_D])
    gg1 = jnp.tanh(g1[:, 2 * _D:3 * _D])
    o1 = jax.nn.sigmoid(g1[:, 3 * _D:])
    r1 = o1 * jnp.tanh(i1 * gg1)
    g2 = jax.lax.dot_general(r1, w2t_ref[...], (((1,), (0,)), ((), ())),
                             preferred_element_type=jnp.float32) + bias2_ref[...]
    i2 = jax.nn.sigmoid(g2[:, :_D])
    gg2 = jnp.tanh(g2[:, 2 * _D:3 * _D])
    o2 = jax.nn.sigmoid(g2[:, 3 * _D:])
    r2 = o2 * jnp.tanh(i2 * gg2)
    acc = (jax.lax.dot_general(jnp.maximum(r2, 0.0), wab_ref[...],
                               (((1,), (0,)), ((), ())),
                               preferred_element_type=jnp.float32)
           + jax.lax.dot_general(jnp.maximum(x_ref[...], 0.0), wc_ref[...],
                                 (((1,), (0,)), ((), ())),
                                 preferred_element_type=jnp.float32))
    out_ref[...] = jnp.tanh(acc + linb_ref[...])


def _tc_c(q0, q1, z2, dis, b, s, t, h1, x, w1a, w1b, bias1, w2t, bias2,
          wab, wc, linb):
    col = pl.BlockSpec((_RB, 1), lambda i: (i, 0))
    mat = pl.BlockSpec((_RB, _D), lambda i: (i, 0))
    row = pl.BlockSpec((1, _D), lambda i: (0, 0))
    w4 = pl.BlockSpec((_D, 4 * _D), lambda i: (0, 0))
    row4 = pl.BlockSpec((1, 4 * _D), lambda i: (0, 0))
    wv = pl.BlockSpec((_D, 1), lambda i: (0, 0))
    sc = pl.BlockSpec((1, 1), lambda i: (0, 0))
    return pl.pallas_call(
        _tc_c_body,
        grid=(_G,),
        in_specs=[mat, mat, mat, col, row, row, row, mat, mat,
                  w4, w4, row4, w4, row4, wv, wv, sc],
        out_specs=col,
        out_shape=jax.ShapeDtypeStruct((_NP, 1), jnp.float32),
    )(q0, q1, z2, dis, b, s, t, h1, x, w1a, w1b, bias1, w2t, bias2,
      wab, wc, linb)


_MESH = plsc.VectorSubcoreMesh(core_axis_name="c", subcore_axis_name="s")


@functools.partial(
    pl.kernel,
    mesh=_MESH,
    out_type=jax.ShapeDtypeStruct((2 * _NP,), jnp.float32),
    scratch_types=[
        pltpu.VMEM((_CH,), jnp.int32),
        pltpu.VMEM((_CH,), jnp.int32),
        pltpu.VMEM((_CH,), jnp.float32),
        pltpu.VMEM((_CH,), jnp.float32),
        pltpu.VMEM((_SLICE,), jnp.float32),
        pltpu.VMEM_SHARED((_NP,), jnp.float32),
        pltpu.SemaphoreType.DMA,
        pltpu.SemaphoreType.DMA,
    ],
)
def _sc_deg(col_hbm, w_hbm, out_hbm, cidx0, cidx1, wch0, wch1, dbuf, acc,
            ds0, ds1):
    cid = lax.axis_index("c")
    sid = lax.axis_index("s")
    wid = cid * 16 + sid
    base = wid * _EPT

    def zero16(i, carry):
        dbuf[pl.ds(i * 16, 16)] = jnp.zeros((16,), jnp.float32)
        return carry

    lax.fori_loop(0, _SLICE // 16, zero16, 0)
    dbuf[pl.ds(_SLICE - 16, 16)] = jnp.zeros((16,), jnp.float32)
    pltpu.sync_copy(dbuf, acc.at[pl.ds(sid * _SLICE, _SLICE)])
    plsc.subcore_barrier()

    pltpu.async_copy(col_hbm.at[pl.ds(base, _CH)], cidx0, ds0)
    pltpu.async_copy(w_hbm.at[pl.ds(base, _CH)], wch0, ds0)

    def pair(p, carry):
        for b, cidx, wch, sem, ocidx, owch, osem in (
                (0, cidx0, wch0, ds0, cidx1, wch1, ds1),
                (1, cidx1, wch1, ds1, cidx0, wch0, ds0)):
            k = 2 * p + b
            pltpu.make_async_copy(
                col_hbm.at[pl.ds(base + k * _CH, _CH)], cidx, sem).wait()
            pltpu.make_async_copy(
                w_hbm.at[pl.ds(base + k * _CH, _CH)], wch, sem).wait()

            @pl.when(k + 1 < _NCH)
            def _():
                pltpu.async_copy(
                    col_hbm.at[pl.ds(base + (k + 1) * _CH, _CH)], ocidx, osem)
                pltpu.async_copy(
                    w_hbm.at[pl.ds(base + (k + 1) * _CH, _CH)], owch, osem)

            pltpu.sync_copy(wch, acc.at[cidx], add=True)
        return carry

    lax.fori_loop(0, _NCH // 2, pair, 0)
    plsc.subcore_barrier()
    pltpu.sync_copy(acc.at[pl.ds(sid * _SLICE, _SLICE)], dbuf)
    pltpu.sync_copy(dbuf, out_hbm.at[pl.ds(cid * _NP + sid * _SLICE, _SLICE)])


@functools.partial(
    pl.kernel,
    mesh=_MESH,
    out_type=jax.ShapeDtypeStruct((2, _NP, _D), jnp.float32),
    scratch_types=[
        pltpu.VMEM((_CH,), jnp.int32),
        pltpu.VMEM((_CH,), jnp.int32),
        pltpu.VMEM((_CH,), jnp.int32),
        pltpu.VMEM((_CH,), jnp.int32),
        pltpu.VMEM((_CH,), jnp.float32),
        pltpu.VMEM((_CH,), jnp.float32),
        pltpu.VMEM((_CH, _D), jnp.float32),
        pltpu.VMEM((_CH, _D), jnp.float32),
        pltpu.VMEM_SHARED((_NP, _D), jnp.float32),
        pltpu.SemaphoreType.DMA,
        pltpu.SemaphoreType.DMA,
        pltpu.SemaphoreType.DMA,
        pltpu.SemaphoreType.DMA,
    ],
)
def _sc_agg(z_hbm, row_hbm, col_hbm, w_hbm, out_hbm,
            ridx0, ridx1, cidx0, cidx1, wch0, wch1, rows0, rows1, acc,
            is0, is1, gs0, gs1):
    cid = lax.axis_index("c")
    sid = lax.axis_index("s")
    ncht = jnp.where(cid == 0, _NC0, _NCH2 - _NC0)
    base = (jnp.where(cid == 0, sid * _NC0, 16 * _NC0 + sid * (_NCH2 - _NC0))
            * _CH)

    def zrow(r, carry):
        for j in range(8):
            rows0[r, pl.ds(j * 16, 16)] = jnp.zeros((16,), jnp.float32)
        return carry

    lax.fori_loop(0, _CH, zrow, 0)
    for p, sz in enumerate((_CH, _CH, _CH, _CH, _SLICE - 4 * _CH)):
        pltpu.sync_copy(rows0.at[pl.ds(0, sz)],
                        acc.at[pl.ds(sid * _SLICE + p * _CH, sz)])
    plsc.subcore_barrier()

    def idx_issue(off, ridx, cidx, wch, isem):
        pltpu.async_copy(row_hbm.at[pl.ds(off, _CH)], ridx, isem)
        pltpu.async_copy(col_hbm.at[pl.ds(off, _CH)], cidx, isem)
        pltpu.async_copy(w_hbm.at[pl.ds(off, _CH)], wch, isem)

    def idx_wait(off, ridx, cidx, wch, isem):
        pltpu.make_async_copy(row_hbm.at[pl.ds(off, _CH)], ridx, isem).wait()
        pltpu.make_async_copy(col_hbm.at[pl.ds(off, _CH)], cidx, isem).wait()
        pltpu.make_async_copy(w_hbm.at[pl.ds(off, _CH)], wch, isem).wait()

    idx_issue(base, ridx0, cidx0, wch0, is0)
    idx_wait(base, ridx0, cidx0, wch0, is0)
    pltpu.async_copy(z_hbm.at[ridx0], rows0, gs0)

    def pair(p, carry):
        for b, ridx, cidx, wch, isem, rows, gs, \
                oridx, ocidx, owch, oisem, orows, ogs in (
                (0, ridx0, cidx0, wch0, is0, rows0, gs0,
                 ridx1, cidx1, wch1, is1, rows1, gs1),
                (1, ridx1, cidx1, wch1, is1, rows1, gs1,
                 ridx0, cidx0, wch0, is0, rows0, gs0)):
            k = 2 * p + b
            nxt_off = base + (k + 1) * _CH

            @pl.when(k + 1 < ncht)
            def _():
                idx_issue(nxt_off, oridx, ocidx, owch, oisem)

            pltpu.make_async_copy(z_hbm.at[ridx], rows, gs).wait()

            def scale(g, c2):
                wv16 = wch[pl.ds(g * 16, 16)]
                for l in range(16):
                    e = g * 16 + l
                    wv = jnp.full((16,), wv16[l])
                    for j in range(8):
                        sl = pl.ds(j * 16, 16)
                        rows[e, sl] = rows[e, sl] * wv
                return c2

            lax.fori_loop(0, _CH // 16, scale, 0)

            @pl.when(k + 1 < ncht)
            def _():
                idx_wait(nxt_off, oridx, ocidx, owch, oisem)
                pltpu.async_copy(z_hbm.at[oridx], orows, ogs)

            pltpu.sync_copy(rows, acc.at[cidx], add=True)
        return carry

    lax.fori_loop(0, ncht // 2, pair, 0)
    plsc.subcore_barrier()
    for p, sz in enumerate((_CH, _CH, _CH, _CH, _SLICE - 4 * _CH)):
        off = sid * _SLICE + p * _CH
        pltpu.sync_copy(acc.at[pl.ds(off, sz)], rows0.at[pl.ds(0, sz)])
        pltpu.sync_copy(rows0.at[pl.ds(0, sz)], out_hbm.at[cid, pl.ds(off, sz)])


def kernel(x, edge_index, edge_weight, W1, b1, bn1_g, bn1_b, bn1_rm, bn1_rv,
           W2, b2, bn2_g, bn2_b, bn2_rm, bn2_rv,
           l1_wih, l1_whh, l1_bih, l1_bhh, l2_wih, l2_whh, l2_bih, l2_bhh,
           lin_w, lin_b):
    f32 = jnp.float32
    row = edge_index[0]
    col = edge_index[1]
    xp = jnp.zeros((_NP, _D), f32).at[:_N].set(x)
    rowp = jnp.zeros((_EP,), jnp.int32).at[:_E].set(row)
    colp = jnp.zeros((_EP,), jnp.int32).at[:_E].set(col)
    wp = jnp.zeros((_EP,), f32).at[:_E].set(edge_weight)

    s1 = (bn1_g / jnp.sqrt(bn1_rv + 1e-5)).reshape(1, _D)
    t1 = (bn1_b - bn1_rm * s1[0]).reshape(1, _D)
    s2 = (bn2_g / jnp.sqrt(bn2_rv + 1e-5)).reshape(1, _D)
    t2 = (bn2_b - bn2_rm * s2[0]).reshape(1, _D)

    w1t = l1_wih.T
    w1a = w1t[:_D]
    w1b = w1t[_D:]
    bias1 = (l1_bih + l1_bhh).reshape(1, 4 * _D)
    w2t = l2_wih.T
    bias2 = (l2_bih + l2_bhh).reshape(1, 4 * _D)
    wab = (lin_w[0, :_D] + lin_w[0, _D:2 * _D]).reshape(_D, 1)
    wc = lin_w[0, 2 * _D:].reshape(_D, 1)
    linb = lin_b.reshape(1, 1)

    deg_p = _sc_deg(colp, wp)
    dis, z1 = _tc_a(deg_p[:_NP].reshape(_NP, 1), deg_p[_NP:].reshape(_NP, 1),
                    xp, W1)

    q1 = _sc_agg(z1, rowp, colp, wp)
    h1, z2 = _tc_b(q1[0], q1[1], z1, dis, b1.reshape(1, _D), s1, t1, W2)

    q2 = _sc_agg(z2, rowp, colp, wp)
    out = _tc_c(q2[0], q2[1], z2, dis, b2.reshape(1, _D), s2, t2, h1, xp,
                w1a, w1b, bias1, w2t, bias2, wab, wc, linb)
    return out[:_N]


# speedup vs baseline: 1.5016x; 1.2163x over previous
import functools

import jax
import jax.numpy as jnp
from jax import lax
from jax.experimental import pallas as pl
from jax.experimental.pallas import tpu as pltpu
from jax.experimental.pallas import tpu_sc as plsc

_N = 10000
_E = 320000
_D = 128
_RB = 512
_G = 20
_NP = _G * _RB

_NT = 32
_CH = 128
_NCH = 80
_EPT = _CH * _NCH
_EP = _NT * _EPT
_SLICE = _NP // 16
_NCH2 = 2 * _NCH
_NC0 = 116


def _tc_a_body(p0_ref, p1_ref, x_ref, w1_ref, dis_ref, z1_ref):
    deg = p0_ref[...] + p1_ref[...] + 1.0
    dis = jax.lax.rsqrt(deg)
    dis_ref[...] = dis
    z1_ref[...] = dis * jax.lax.dot_general(
        x_ref[...], w1_ref[...], (((1,), (0,)), ((), ())),
        preferred_element_type=jnp.float32)


def _tc_a(p0, p1, x, w1):
    col = pl.BlockSpec((_RB, 1), lambda i: (i, 0))
    mat = pl.BlockSpec((_RB, _D), lambda i: (i, 0))
    wsp = pl.BlockSpec((_D, _D), lambda i: (0, 0))
    return pl.pallas_call(
        _tc_a_body,
        grid=(_G,),
        in_specs=[col, col, mat, wsp],
        out_specs=[col, mat],
        out_shape=[jax.ShapeDtypeStruct((_NP, 1), jnp.float32),
                   jax.ShapeDtypeStruct((_NP, _D), jnp.float32)],
    )(p0, p1, x, w1)


def _tc_b_body(q0_ref, q1_ref, z_ref, dis_ref, b_ref, s_ref, t_ref, w2_ref,
               h_ref, z2_ref):
    dis = dis_ref[...]
    gcn = dis * (q0_ref[...] + q1_ref[...] + z_ref[...]) + b_ref[...]
    h = jnp.maximum(gcn, 0.0) * s_ref[...] + t_ref[...]
    h_ref[...] = h
    z2_ref[...] = dis * jax.lax.dot_general(
        h, w2_ref[...], (((1,), (0,)), ((), ())),
        preferred_element_type=jnp.float32)


def _tc_b(q0, q1, z, dis, b, s, t, w2):
    col = pl.BlockSpec((_RB, 1), lambda i: (i, 0))
    mat = pl.BlockSpec((_RB, _D), lambda i: (i, 0))
    row = pl.BlockSpec((1, _D), lambda i: (0, 0))
    wsp = pl.BlockSpec((_D, _D), lambda i: (0, 0))
    return pl.pallas_call(
        _tc_b_body,
        grid=(_G,),
        in_specs=[mat, mat, mat, col, row, row, row, wsp],
        out_specs=[mat, mat],
        out_shape=[jax.ShapeDtypeStruct((_NP, _D), jnp.float32),
                   jax.ShapeDtypeStruct((_NP, _D), jnp.float32)],
    )(q0, q1, z, dis, b, s, t, w2)


def _tc_c_body(q0_ref, q1_ref, z2_ref, dis_ref, b_ref, s_ref, t_ref,
               h1_ref, x_ref, w1a_ref, w1b_ref, bias1_ref, w2t_ref, bias2_ref,
               wab_ref, wc_ref, linb_ref, out_ref):
    dis = dis_ref[...]
    gcn = dis * (q0_ref[...] + q1_ref[...] + z2_ref[...]) + b_ref[...]
    h2 = jnp.maximum(gcn, 0.0) * s_ref[...] + t_ref[...]
    h1 = h1_ref[...]
    g1 = (jax.lax.dot_general(h1, w1a_ref[...], (((1,), (0,)), ((), ())),
                              preferred_element_type=jnp.float32)
          + jax.lax.dot_general(h2, w1b_ref[...], (((1,), (0,)), ((), ())),
                                preferred_element_type=jnp.float32)
          + bias1_ref[...])
    i1 = jax.nn.sigmoid(g1[:, :_D])
    gg1 = jnp.tanh(g1[:, 2 * _D:3 * _D])
    o1 = jax.nn.sigmoid(g1[:, 3 * _D:])
    r1 = o1 * jnp.tanh(i1 * gg1)
    g2 = jax.lax.dot_general(r1, w2t_ref[...], (((1,), (0,)), ((), ())),
                             preferred_element_type=jnp.float32) + bias2_ref[...]
    i2 = jax.nn.sigmoid(g2[:, :_D])
    gg2 = jnp.tanh(g2[:, 2 * _D:3 * _D])
    o2 = jax.nn.sigmoid(g2[:, 3 * _D:])
    r2 = o2 * jnp.tanh(i2 * gg2)
    acc = (jax.lax.dot_general(jnp.maximum(r2, 0.0), wab_ref[...],
                               (((1,), (0,)), ((), ())),
                               preferred_element_type=jnp.float32)
           + jax.lax.dot_general(jnp.maximum(x_ref[...], 0.0), wc_ref[...],
                                 (((1,), (0,)), ((), ())),
                                 preferred_element_type=jnp.float32))
    out_ref[...] = jnp.tanh(acc + linb_ref[...])


def _tc_c(q0, q1, z2, dis, b, s, t, h1, x, w1a, w1b, bias1, w2t, bias2,
          wab, wc, linb):
    col = pl.BlockSpec((_RB, 1), lambda i: (i, 0))
    mat = pl.BlockSpec((_RB, _D), lambda i: (i, 0))
    row = pl.BlockSpec((1, _D), lambda i: (0, 0))
    w4 = pl.BlockSpec((_D, 4 * _D), lambda i: (0, 0))
    row4 = pl.BlockSpec((1, 4 * _D), lambda i: (0, 0))
    wv = pl.BlockSpec((_D, 1), lambda i: (0, 0))
    sc = pl.BlockSpec((1, 1), lambda i: (0, 0))
    return pl.pallas_call(
        _tc_c_body,
        grid=(_G,),
        in_specs=[mat, mat, mat, col, row, row, row, mat, mat,
                  w4, w4, row4, w4, row4, wv, wv, sc],
        out_specs=col,
        out_shape=jax.ShapeDtypeStruct((_NP, 1), jnp.float32),
    )(q0, q1, z2, dis, b, s, t, h1, x, w1a, w1b, bias1, w2t, bias2,
      wab, wc, linb)


_MESH = plsc.VectorSubcoreMesh(core_axis_name="c", subcore_axis_name="s")


@functools.partial(
    pl.kernel,
    mesh=_MESH,
    out_type=jax.ShapeDtypeStruct((2 * _NP,), jnp.float32),
    scratch_types=[
        pltpu.VMEM((_CH,), jnp.int32),
        pltpu.VMEM((_CH,), jnp.int32),
        pltpu.VMEM((_CH,), jnp.float32),
        pltpu.VMEM((_CH,), jnp.float32),
        pltpu.VMEM((_SLICE,), jnp.float32),
        pltpu.VMEM_SHARED((_NP,), jnp.float32),
        pltpu.SemaphoreType.DMA,
        pltpu.SemaphoreType.DMA,
    ],
)
def _sc_deg(col_hbm, w_hbm, out_hbm, cidx0, cidx1, wch0, wch1, dbuf, acc,
            ds0, ds1):
    cid = lax.axis_index("c")
    sid = lax.axis_index("s")
    wid = cid * 16 + sid
    base = wid * _EPT

    def zero16(i, carry):
        dbuf[pl.ds(i * 16, 16)] = jnp.zeros((16,), jnp.float32)
        return carry

    lax.fori_loop(0, _SLICE // 16, zero16, 0)
    pltpu.sync_copy(dbuf, acc.at[pl.ds(sid * _SLICE, _SLICE)])
    plsc.subcore_barrier()

    pltpu.async_copy(col_hbm.at[pl.ds(base, _CH)], cidx0, ds0)
    pltpu.async_copy(w_hbm.at[pl.ds(base, _CH)], wch0, ds0)

    def pair(p, carry):
        for b, cidx, wch, sem, ocidx, owch, osem in (
                (0, cidx0, wch0, ds0, cidx1, wch1, ds1),
                (1, cidx1, wch1, ds1, cidx0, wch0, ds0)):
            k = 2 * p + b
            pltpu.make_async_copy(
                col_hbm.at[pl.ds(base + k * _CH, _CH)], cidx, sem).wait()
            pltpu.make_async_copy(
                w_hbm.at[pl.ds(base + k * _CH, _CH)], wch, sem).wait()

            @pl.when(k + 1 < _NCH)
            def _():
                pltpu.async_copy(
                    col_hbm.at[pl.ds(base + (k + 1) * _CH, _CH)], ocidx, osem)
                pltpu.async_copy(
                    w_hbm.at[pl.ds(base + (k + 1) * _CH, _CH)], owch, osem)

            pltpu.sync_copy(wch, acc.at[cidx], add=True)
        return carry

    lax.fori_loop(0, _NCH // 2, pair, 0)
    plsc.subcore_barrier()
    pltpu.sync_copy(acc.at[pl.ds(sid * _SLICE, _SLICE)], dbuf)
    pltpu.sync_copy(dbuf, out_hbm.at[pl.ds(cid * _NP + sid * _SLICE, _SLICE)])


@functools.partial(
    pl.kernel,
    mesh=_MESH,
    out_type=jax.ShapeDtypeStruct((2, _NP, _D), jnp.float32),
    scratch_types=[
        pltpu.VMEM((_CH,), jnp.int32),
        pltpu.VMEM((_CH,), jnp.int32),
        pltpu.VMEM((_CH,), jnp.int32),
        pltpu.VMEM((_CH,), jnp.int32),
        pltpu.VMEM((_CH,), jnp.float32),
        pltpu.VMEM((_CH,), jnp.float32),
        pltpu.VMEM((_CH, _D), jnp.float32),
        pltpu.VMEM((_CH, _D), jnp.float32),
        pltpu.VMEM_SHARED((_NP, _D), jnp.float32),
        pltpu.SemaphoreType.DMA,
        pltpu.SemaphoreType.DMA,
        pltpu.SemaphoreType.DMA,
        pltpu.SemaphoreType.DMA,
    ],
)
def _sc_agg(z_hbm, row_hbm, col_hbm, w_hbm, out_hbm,
            ridx0, ridx1, cidx0, cidx1, wch0, wch1, rows0, rows1, acc,
            is0, is1, gs0, gs1):
    cid = lax.axis_index("c")
    sid = lax.axis_index("s")
    ncht = jnp.where(cid == 0, _NC0, _NCH2 - _NC0)
    base = (jnp.where(cid == 0, sid * _NC0, 16 * _NC0 + sid * (_NCH2 - _NC0))
            * _CH)

    def zrow(r, carry):
        for j in range(8):
            rows0[r, pl.ds(j * 16, 16)] = jnp.zeros((16,), jnp.float32)
        return carry

    lax.fori_loop(0, _CH, zrow, 0)
    for p in range(_SLICE // _CH):
        pltpu.sync_copy(rows0, acc.at[pl.ds(sid * _SLICE + p * _CH, _CH)])
    plsc.subcore_barrier()

    def idx_issue(off, ridx, cidx, wch, isem):
        pltpu.async_copy(row_hbm.at[pl.ds(off, _CH)], ridx, isem)
        pltpu.async_copy(col_hbm.at[pl.ds(off, _CH)], cidx, isem)
        pltpu.async_copy(w_hbm.at[pl.ds(off, _CH)], wch, isem)

    def idx_wait(off, ridx, cidx, wch, isem):
        pltpu.make_async_copy(row_hbm.at[pl.ds(off, _CH)], ridx, isem).wait()
        pltpu.make_async_copy(col_hbm.at[pl.ds(off, _CH)], cidx, isem).wait()
        pltpu.make_async_copy(w_hbm.at[pl.ds(off, _CH)], wch, isem).wait()

    idx_issue(base, ridx0, cidx0, wch0, is0)
    idx_wait(base, ridx0, cidx0, wch0, is0)
    pltpu.async_copy(z_hbm.at[ridx0], rows0, gs0)

    def pair(p, carry):
        for b, ridx, cidx, wch, isem, rows, gs, \
                oridx, ocidx, owch, oisem, orows, ogs in (
                (0, ridx0, cidx0, wch0, is0, rows0, gs0,
                 ridx1, cidx1, wch1, is1, rows1, gs1),
                (1, ridx1, cidx1, wch1, is1, rows1, gs1,
                 ridx0, cidx0, wch0, is0, rows0, gs0)):
            k = 2 * p + b
            nxt_off = base + (k + 1) * _CH

            @pl.when(k + 1 < ncht)
            def _():
                idx_issue(nxt_off, oridx, ocidx, owch, oisem)

            pltpu.make_async_copy(z_hbm.at[ridx], rows, gs).wait()

            def scale(g, c2):
                wv16 = wch[pl.ds(g * 16, 16)]
                for l in range(16):
                    e = g * 16 + l
                    wv = jnp.full((16,), wv16[l])
                    for j in range(8):
                        sl = pl.ds(j * 16, 16)
                        rows[e, sl] = rows[e, sl] * wv
                return c2

            lax.fori_loop(0, _CH // 16, scale, 0)

            @pl.when(k + 1 < ncht)
            def _():
                idx_wait(nxt_off, oridx, ocidx, owch, oisem)
                pltpu.async_copy(z_hbm.at[oridx], orows, ogs)

            pltpu.sync_copy(rows, acc.at[cidx], add=True)
        return carry

    lax.fori_loop(0, ncht // 2, pair, 0)
    plsc.subcore_barrier()
    for p in range(_SLICE // _CH):
        off = sid * _SLICE + p * _CH
        pltpu.sync_copy(acc.at[pl.ds(off, _CH)], rows0)
        pltpu.sync_copy(rows0, out_hbm.at[cid, pl.ds(off, _CH)])


def kernel(x, edge_index, edge_weight, W1, b1, bn1_g, bn1_b, bn1_rm, bn1_rv,
           W2, b2, bn2_g, bn2_b, bn2_rm, bn2_rv,
           l1_wih, l1_whh, l1_bih, l1_bhh, l2_wih, l2_whh, l2_bih, l2_bhh,
           lin_w, lin_b):
    f32 = jnp.float32
    row = edge_index[0]
    col = edge_index[1]
    xp = jnp.zeros((_NP, _D), f32).at[:_N].set(x)
    rowp = jnp.zeros((_EP,), jnp.int32).at[:_E].set(row)
    colp = jnp.zeros((_EP,), jnp.int32).at[:_E].set(col)
    wp = jnp.zeros((_EP,), f32).at[:_E].set(edge_weight)

    s1 = (bn1_g / jnp.sqrt(bn1_rv + 1e-5)).reshape(1, _D)
    t1 = (bn1_b - bn1_rm * s1[0]).reshape(1, _D)
    s2 = (bn2_g / jnp.sqrt(bn2_rv + 1e-5)).reshape(1, _D)
    t2 = (bn2_b - bn2_rm * s2[0]).reshape(1, _D)

    w1t = l1_wih.T
    w1a = w1t[:_D]
    w1b = w1t[_D:]
    bias1 = (l1_bih + l1_bhh).reshape(1, 4 * _D)
    w2t = l2_wih.T
    bias2 = (l2_bih + l2_bhh).reshape(1, 4 * _D)
    wab = (lin_w[0, :_D] + lin_w[0, _D:2 * _D]).reshape(_D, 1)
    wc = lin_w[0, 2 * _D:].reshape(_D, 1)
    linb = lin_b.reshape(1, 1)

    deg_p = _sc_deg(colp, wp)
    dis, z1 = _tc_a(deg_p[:_NP].reshape(_NP, 1), deg_p[_NP:].reshape(_NP, 1),
                    xp, W1)

    q1 = _sc_agg(z1, rowp, colp, wp)
    h1, z2 = _tc_b(q1[0], q1[1], z1, dis, b1.reshape(1, _D), s1, t1, W2)

    q2 = _sc_agg(z2, rowp, colp, wp)
    out = _tc_c(q2[0], q2[1], z2, dis, b2.reshape(1, _D), s2, t2, h1, xp,
                w1a, w1b, bias1, w2t, bias2, wab, wc, linb)
    return out[:_N]


# speedup vs baseline: 1.5313x; 1.0198x over previous
import functools

import jax
import jax.numpy as jnp
from jax import lax
from jax.experimental import pallas as pl
from jax.experimental.pallas import tpu as pltpu
from jax.experimental.pallas import tpu_sc as plsc

_N = 10000
_E = 320000
_D = 128
_RB = 1024
_G = 10
_NP = _G * _RB

_NT = 32
_CH = 128
_NCH = 80
_EPT = _CH * _NCH
_EP = _NT * _EPT
_SLICE = _NP // 16
_NCH2 = 2 * _NCH
_NC0 = 116


def _tc_a_body(p0_ref, p1_ref, x_ref, w1_ref, dis_ref, z1_ref):
    deg = p0_ref[...] + p1_ref[...] + 1.0
    dis = jax.lax.rsqrt(deg)
    dis_ref[...] = dis
    z1_ref[...] = dis * jax.lax.dot_general(
        x_ref[...], w1_ref[...], (((1,), (0,)), ((), ())),
        preferred_element_type=jnp.float32)


def _tc_a(p0, p1, x, w1):
    col = pl.BlockSpec((_RB, 1), lambda i: (i, 0))
    mat = pl.BlockSpec((_RB, _D), lambda i: (i, 0))
    wsp = pl.BlockSpec((_D, _D), lambda i: (0, 0))
    return pl.pallas_call(
        _tc_a_body,
        grid=(_G,),
        in_specs=[col, col, mat, wsp],
        out_specs=[col, mat],
        out_shape=[jax.ShapeDtypeStruct((_NP, 1), jnp.float32),
                   jax.ShapeDtypeStruct((_NP, _D), jnp.float32)],
    )(p0, p1, x, w1)


def _tc_b_body(q0_ref, q1_ref, z_ref, dis_ref, b_ref, s_ref, t_ref, w2_ref,
               h_ref, z2_ref):
    dis = dis_ref[...]
    gcn = dis * (q0_ref[...] + q1_ref[...] + z_ref[...]) + b_ref[...]
    h = jnp.maximum(gcn, 0.0) * s_ref[...] + t_ref[...]
    h_ref[...] = h
    z2_ref[...] = dis * jax.lax.dot_general(
        h, w2_ref[...], (((1,), (0,)), ((), ())),
        preferred_element_type=jnp.float32)


def _tc_b(q0, q1, z, dis, b, s, t, w2):
    col = pl.BlockSpec((_RB, 1), lambda i: (i, 0))
    mat = pl.BlockSpec((_RB, _D), lambda i: (i, 0))
    row = pl.BlockSpec((1, _D), lambda i: (0, 0))
    wsp = pl.BlockSpec((_D, _D), lambda i: (0, 0))
    return pl.pallas_call(
        _tc_b_body,
        grid=(_G,),
        in_specs=[mat, mat, mat, col, row, row, row, wsp],
        out_specs=[mat, mat],
        out_shape=[jax.ShapeDtypeStruct((_NP, _D), jnp.float32),
                   jax.ShapeDtypeStruct((_NP, _D), jnp.float32)],
    )(q0, q1, z, dis, b, s, t, w2)


def _tc_c_body(q0_ref, q1_ref, z2_ref, dis_ref, b_ref, s_ref, t_ref,
               h1_ref, x_ref, w1a_ref, w1b_ref, bias1_ref, w2t_ref, bias2_ref,
               wab_ref, wc_ref, linb_ref, out_ref):
    dis = dis_ref[...]
    gcn = dis * (q0_ref[...] + q1_ref[...] + z2_ref[...]) + b_ref[...]
    h2 = jnp.maximum(gcn, 0.0) * s_ref[...] + t_ref[...]
    h1 = h1_ref[...]
    g1 = (jax.lax.dot_general(h1, w1a_ref[...], (((1,), (0,)), ((), ())),
                              preferred_element_type=jnp.float32)
          + jax.lax.dot_general(h2, w1b_ref[...], (((1,), (0,)), ((), ())),
                                preferred_element_type=jnp.float32)
          + bias1_ref[...])
    i1 = jax.nn.sigmoid(g1[:, :_D])
    gg1 = jnp.tanh(g1[:, 2 * _D:3 * _D])
    o1 = jax.nn.sigmoid(g1[:, 3 * _D:])
    r1 = o1 * jnp.tanh(i1 * gg1)
    g2 = jax.lax.dot_general(r1, w2t_ref[...], (((1,), (0,)), ((), ())),
                             preferred_element_type=jnp.float32) + bias2_ref[...]
    i2 = jax.nn.sigmoid(g2[:, :_D])
    gg2 = jnp.tanh(g2[:, 2 * _D:3 * _D])
    o2 = jax.nn.sigmoid(g2[:, 3 * _D:])
    r2 = o2 * jnp.tanh(i2 * gg2)
    acc = (jax.lax.dot_general(jnp.maximum(r2, 0.0), wab_ref[...],
                               (((1,), (0,)), ((), ())),
                               preferred_element_type=jnp.float32)
           + jax.lax.dot_general(jnp.maximum(x_ref[...], 0.0), wc_ref[...],
                                 (((1,), (0,)), ((), ())),
                                 preferred_element_type=jnp.float32))
    out_ref[...] = jnp.tanh(acc + linb_ref[...])


def _tc_c(q0, q1, z2, dis, b, s, t, h1, x, w1a, w1b, bias1, w2t, bias2,
          wab, wc, linb):
    col = pl.BlockSpec((_RB, 1), lambda i: (i, 0))
    mat = pl.BlockSpec((_RB, _D), lambda i: (i, 0))
    row = pl.BlockSpec((1, _D), lambda i: (0, 0))
    w4 = pl.BlockSpec((_D, 4 * _D), lambda i: (0, 0))
    row4 = pl.BlockSpec((1, 4 * _D), lambda i: (0, 0))
    wv = pl.BlockSpec((_D, 1), lambda i: (0, 0))
    sc = pl.BlockSpec((1, 1), lambda i: (0, 0))
    return pl.pallas_call(
        _tc_c_body,
        grid=(_G,),
        in_specs=[mat, mat, mat, col, row, row, row, mat, mat,
                  w4, w4, row4, w4, row4, wv, wv, sc],
        out_specs=col,
        out_shape=jax.ShapeDtypeStruct((_NP, 1), jnp.float32),
    )(q0, q1, z2, dis, b, s, t, h1, x, w1a, w1b, bias1, w2t, bias2,
      wab, wc, linb)


_MESH = plsc.VectorSubcoreMesh(core_axis_name="c", subcore_axis_name="s")


@functools.partial(
    pl.kernel,
    mesh=_MESH,
    out_type=jax.ShapeDtypeStruct((2 * _NP,), jnp.float32),
    scratch_types=[
        pltpu.VMEM((_CH,), jnp.int32),
        pltpu.VMEM((_CH,), jnp.int32),
        pltpu.VMEM((_CH,), jnp.float32),
        pltpu.VMEM((_CH,), jnp.float32),
        pltpu.VMEM((_SLICE,), jnp.float32),
        pltpu.VMEM_SHARED((_NP,), jnp.float32),
        pltpu.SemaphoreType.DMA,
        pltpu.SemaphoreType.DMA,
    ],
)
def _sc_deg(col_hbm, w_hbm, out_hbm, cidx0, cidx1, wch0, wch1, dbuf, acc,
            ds0, ds1):
    cid = lax.axis_index("c")
    sid = lax.axis_index("s")
    wid = cid * 16 + sid
    base = wid * _EPT

    def zero16(i, carry):
        dbuf[pl.ds(i * 16, 16)] = jnp.zeros((16,), jnp.float32)
        return carry

    lax.fori_loop(0, _SLICE // 16, zero16, 0)
    pltpu.sync_copy(dbuf, acc.at[pl.ds(sid * _SLICE, _SLICE)])
    plsc.subcore_barrier()

    pltpu.async_copy(col_hbm.at[pl.ds(base, _CH)], cidx0, ds0)
    pltpu.async_copy(w_hbm.at[pl.ds(base, _CH)], wch0, ds0)

    def pair(p, carry):
        for b, cidx, wch, sem, ocidx, owch, osem in (
                (0, cidx0, wch0, ds0, cidx1, wch1, ds1),
                (1, cidx1, wch1, ds1, cidx0, wch0, ds0)):
            k = 2 * p + b
            pltpu.make_async_copy(
                col_hbm.at[pl.ds(base + k * _CH, _CH)], cidx, sem).wait()
            pltpu.make_async_copy(
                w_hbm.at[pl.ds(base + k * _CH, _CH)], wch, sem).wait()

            @pl.when(k + 1 < _NCH)
            def _():
                pltpu.async_copy(
                    col_hbm.at[pl.ds(base + (k + 1) * _CH, _CH)], ocidx, osem)
                pltpu.async_copy(
                    w_hbm.at[pl.ds(base + (k + 1) * _CH, _CH)], owch, osem)

            pltpu.sync_copy(wch, acc.at[cidx], add=True)
        return carry

    lax.fori_loop(0, _NCH // 2, pair, 0)
    plsc.subcore_barrier()
    pltpu.sync_copy(acc.at[pl.ds(sid * _SLICE, _SLICE)], dbuf)
    pltpu.sync_copy(dbuf, out_hbm.at[pl.ds(cid * _NP + sid * _SLICE, _SLICE)])


@functools.partial(
    pl.kernel,
    mesh=_MESH,
    out_type=jax.ShapeDtypeStruct((2, _NP, _D), jnp.float32),
    scratch_types=[
        pltpu.VMEM((_CH,), jnp.int32),
        pltpu.VMEM((_CH,), jnp.int32),
        pltpu.VMEM((_CH,), jnp.int32),
        pltpu.VMEM((_CH,), jnp.int32),
        pltpu.VMEM((_CH,), jnp.float32),
        pltpu.VMEM((_CH,), jnp.float32),
        pltpu.VMEM((_CH, _D), jnp.float32),
        pltpu.VMEM((_CH, _D), jnp.float32),
        pltpu.VMEM_SHARED((_NP, _D), jnp.float32),
        pltpu.SemaphoreType.DMA,
        pltpu.SemaphoreType.DMA,
        pltpu.SemaphoreType.DMA,
        pltpu.SemaphoreType.DMA,
    ],
)
def _sc_agg(z_hbm, row_hbm, col_hbm, w_hbm, out_hbm,
            ridx0, ridx1, cidx0, cidx1, wch0, wch1, rows0, rows1, acc,
            is0, is1, gs0, gs1):
    cid = lax.axis_index("c")
    sid = lax.axis_index("s")
    ncht = jnp.where(cid == 0, _NC0, _NCH2 - _NC0)
    base = (jnp.where(cid == 0, sid * _NC0, 16 * _NC0 + sid * (_NCH2 - _NC0))
            * _CH)

    def zrow(r, carry):
        for j in range(8):
            rows0[r, pl.ds(j * 16, 16)] = jnp.zeros((16,), jnp.float32)
        return carry

    lax.fori_loop(0, _CH, zrow, 0)
    for p in range(_SLICE // _CH):
        pltpu.sync_copy(rows0, acc.at[pl.ds(sid * _SLICE + p * _CH, _CH)])
    plsc.subcore_barrier()

    def idx_issue(off, ridx, cidx, wch, isem):
        pltpu.async_copy(row_hbm.at[pl.ds(off, _CH)], ridx, isem)
        pltpu.async_copy(col_hbm.at[pl.ds(off, _CH)], cidx, isem)
        pltpu.async_copy(w_hbm.at[pl.ds(off, _CH)], wch, isem)

    def idx_wait(off, ridx, cidx, wch, isem):
        pltpu.make_async_copy(row_hbm.at[pl.ds(off, _CH)], ridx, isem).wait()
        pltpu.make_async_copy(col_hbm.at[pl.ds(off, _CH)], cidx, isem).wait()
        pltpu.make_async_copy(w_hbm.at[pl.ds(off, _CH)], wch, isem).wait()

    idx_issue(base, ridx0, cidx0, wch0, is0)
    idx_wait(base, ridx0, cidx0, wch0, is0)
    pltpu.async_copy(z_hbm.at[ridx0], rows0, gs0)

    def pair(p, carry):
        for b, ridx, cidx, wch, isem, rows, gs, \
                oridx, ocidx, owch, oisem, orows, ogs in (
                (0, ridx0, cidx0, wch0, is0, rows0, gs0,
                 ridx1, cidx1, wch1, is1, rows1, gs1),
                (1, ridx1, cidx1, wch1, is1, rows1, gs1,
                 ridx0, cidx0, wch0, is0, rows0, gs0)):
            k = 2 * p + b
            nxt_off = base + (k + 1) * _CH

            @pl.when(k + 1 < ncht)
            def _():
                idx_issue(nxt_off, oridx, ocidx, owch, oisem)

            pltpu.make_async_copy(z_hbm.at[ridx], rows, gs).wait()

            def scale(g, c2):
                wv16 = wch[pl.ds(g * 16, 16)]
                for l in range(16):
                    e = g * 16 + l
                    wv = jnp.full((16,), wv16[l])
                    for j in range(8):
                        sl = pl.ds(j * 16, 16)
                        rows[e, sl] = rows[e, sl] * wv
                return c2

            lax.fori_loop(0, _CH // 16, scale, 0)

            @pl.when(k + 1 < ncht)
            def _():
                idx_wait(nxt_off, oridx, ocidx, owch, oisem)
                pltpu.async_copy(z_hbm.at[oridx], orows, ogs)

            pltpu.sync_copy(rows, acc.at[cidx], add=True)
        return carry

    lax.fori_loop(0, ncht // 2, pair, 0)
    plsc.subcore_barrier()
    for p in range(_SLICE // _CH):
        off = sid * _SLICE + p * _CH
        pltpu.sync_copy(acc.at[pl.ds(off, _CH)], rows0)
        pltpu.sync_copy(rows0, out_hbm.at[cid, pl.ds(off, _CH)])


def kernel(x, edge_index, edge_weight, W1, b1, bn1_g, bn1_b, bn1_rm, bn1_rv,
           W2, b2, bn2_g, bn2_b, bn2_rm, bn2_rv,
           l1_wih, l1_whh, l1_bih, l1_bhh, l2_wih, l2_whh, l2_bih, l2_bhh,
           lin_w, lin_b):
    f32 = jnp.float32
    row = edge_index[0]
    col = edge_index[1]
    xp = jnp.zeros((_NP, _D), f32).at[:_N].set(x)
    rowp = jnp.zeros((_EP,), jnp.int32).at[:_E].set(row)
    colp = jnp.zeros((_EP,), jnp.int32).at[:_E].set(col)
    wp = jnp.zeros((_EP,), f32).at[:_E].set(edge_weight)

    s1 = (bn1_g / jnp.sqrt(bn1_rv + 1e-5)).reshape(1, _D)
    t1 = (bn1_b - bn1_rm * s1[0]).reshape(1, _D)
    s2 = (bn2_g / jnp.sqrt(bn2_rv + 1e-5)).reshape(1, _D)
    t2 = (bn2_b - bn2_rm * s2[0]).reshape(1, _D)

    w1t = l1_wih.T
    w1a = w1t[:_D]
    w1b = w1t[_D:]
    bias1 = (l1_bih + l1_bhh).reshape(1, 4 * _D)
    w2t = l2_wih.T
    bias2 = (l2_bih + l2_bhh).reshape(1, 4 * _D)
    wab = (lin_w[0, :_D] + lin_w[0, _D:2 * _D]).reshape(_D, 1)
    wc = lin_w[0, 2 * _D:].reshape(_D, 1)
    linb = lin_b.reshape(1, 1)

    deg_p = _sc_deg(colp, wp)
    dis, z1 = _tc_a(deg_p[:_NP].reshape(_NP, 1), deg_p[_NP:].reshape(_NP, 1),
                    xp, W1)

    q1 = _sc_agg(z1, rowp, colp, wp)
    h1, z2 = _tc_b(q1[0], q1[1], z1, dis, b1.reshape(1, _D), s1, t1, W2)

    q2 = _sc_agg(z2, rowp, colp, wp)
    out = _tc_c(q2[0], q2[1], z2, dis, b2.reshape(1, _D), s2, t2, h1, xp,
                w1a, w1b, bias1, w2t, bias2, wab, wc, linb)
    return out[:_N]


# speedup vs baseline: 1.5362x; 1.0032x over previous
import functools

import jax
import jax.numpy as jnp
from jax import lax
from jax.experimental import pallas as pl
from jax.experimental.pallas import tpu as pltpu
from jax.experimental.pallas import tpu_sc as plsc

_N = 10000
_E = 320000
_D = 128
_RB = 1024
_G = 10
_NP = _G * _RB

_NT = 32
_CH = 128
_NCH = 80
_EPT = _CH * _NCH
_EP = _NT * _EPT
_SLICE = _NP // 16
_NCH2 = 2 * _NCH
_NC0 = 116


def _tc_a_body(p0_ref, p1_ref, x_ref, w1_ref, dis_ref, z1_ref):
    deg = p0_ref[...] + p1_ref[...] + 1.0
    dis = jax.lax.rsqrt(deg)
    dis_ref[...] = dis
    z1_ref[...] = dis * jax.lax.dot_general(
        x_ref[...], w1_ref[...], (((1,), (0,)), ((), ())),
        preferred_element_type=jnp.float32)


def _tc_a(p0, p1, x, w1):
    col = pl.BlockSpec((_RB, 1), lambda i: (i, 0))
    mat = pl.BlockSpec((_RB, _D), lambda i: (i, 0))
    wsp = pl.BlockSpec((_D, _D), lambda i: (0, 0))
    return pl.pallas_call(
        _tc_a_body,
        grid=(_G,),
        in_specs=[col, col, mat, wsp],
        out_specs=[col, mat],
        out_shape=[jax.ShapeDtypeStruct((_NP, 1), jnp.float32),
                   jax.ShapeDtypeStruct((_NP, _D), jnp.float32)],
    )(p0, p1, x, w1)


def _tc_b_body(q0_ref, q1_ref, z_ref, dis_ref, b_ref, s_ref, t_ref, w2_ref,
               h_ref, z2_ref):
    dis = dis_ref[...]
    gcn = dis * (q0_ref[...] + q1_ref[...] + z_ref[...]) + b_ref[...]
    h = jnp.maximum(gcn, 0.0) * s_ref[...] + t_ref[...]
    h_ref[...] = h
    z2_ref[...] = dis * jax.lax.dot_general(
        h, w2_ref[...], (((1,), (0,)), ((), ())),
        preferred_element_type=jnp.float32)


def _tc_b(q0, q1, z, dis, b, s, t, w2):
    col = pl.BlockSpec((_RB, 1), lambda i: (i, 0))
    mat = pl.BlockSpec((_RB, _D), lambda i: (i, 0))
    row = pl.BlockSpec((1, _D), lambda i: (0, 0))
    wsp = pl.BlockSpec((_D, _D), lambda i: (0, 0))
    return pl.pallas_call(
        _tc_b_body,
        grid=(_G,),
        in_specs=[mat, mat, mat, col, row, row, row, wsp],
        out_specs=[mat, mat],
        out_shape=[jax.ShapeDtypeStruct((_NP, _D), jnp.float32),
                   jax.ShapeDtypeStruct((_NP, _D), jnp.float32)],
    )(q0, q1, z, dis, b, s, t, w2)


def _tc_c_body(q0_ref, q1_ref, z2_ref, dis_ref, b_ref, s_ref, t_ref,
               h1_ref, x_ref, w1a_ref, w1b_ref, bias1_ref, w2t_ref, bias2_ref,
               wab_ref, wc_ref, linb_ref, out_ref):
    dis = dis_ref[...]
    gcn = dis * (q0_ref[...] + q1_ref[...] + z2_ref[...]) + b_ref[...]
    h2 = jnp.maximum(gcn, 0.0) * s_ref[...] + t_ref[...]
    h1 = h1_ref[...]
    g1 = (jax.lax.dot_general(h1, w1a_ref[...], (((1,), (0,)), ((), ())),
                              preferred_element_type=jnp.float32)
          + jax.lax.dot_general(h2, w1b_ref[...], (((1,), (0,)), ((), ())),
                                preferred_element_type=jnp.float32)
          + bias1_ref[...])
    i1 = jax.nn.sigmoid(g1[:, :_D])
    gg1 = jnp.tanh(g1[:, 2 * _D:3 * _D])
    o1 = jax.nn.sigmoid(g1[:, 3 * _D:])
    r1 = o1 * jnp.tanh(i1 * gg1)
    g2 = jax.lax.dot_general(r1, w2t_ref[...], (((1,), (0,)), ((), ())),
                             preferred_element_type=jnp.float32) + bias2_ref[...]
    i2 = jax.nn.sigmoid(g2[:, :_D])
    gg2 = jnp.tanh(g2[:, 2 * _D:3 * _D])
    o2 = jax.nn.sigmoid(g2[:, 3 * _D:])
    r2 = o2 * jnp.tanh(i2 * gg2)
    acc = (jax.lax.dot_general(jnp.maximum(r2, 0.0), wab_ref[...],
                               (((1,), (0,)), ((), ())),
                               preferred_element_type=jnp.float32)
           + jax.lax.dot_general(jnp.maximum(x_ref[...], 0.0), wc_ref[...],
                                 (((1,), (0,)), ((), ())),
                                 preferred_element_type=jnp.float32))
    out_ref[...] = jnp.tanh(acc + linb_ref[...])


def _tc_c(q0, q1, z2, dis, b, s, t, h1, x, w1a, w1b, bias1, w2t, bias2,
          wab, wc, linb):
    col = pl.BlockSpec((_RB, 1), lambda i: (i, 0))
    mat = pl.BlockSpec((_RB, _D), lambda i: (i, 0))
    row = pl.BlockSpec((1, _D), lambda i: (0, 0))
    w4 = pl.BlockSpec((_D, 4 * _D), lambda i: (0, 0))
    row4 = pl.BlockSpec((1, 4 * _D), lambda i: (0, 0))
    wv = pl.BlockSpec((_D, 1), lambda i: (0, 0))
    sc = pl.BlockSpec((1, 1), lambda i: (0, 0))
    return pl.pallas_call(
        _tc_c_body,
        grid=(_G,),
        in_specs=[mat, mat, mat, col, row, row, row, mat, mat,
                  w4, w4, row4, w4, row4, wv, wv, sc],
        out_specs=col,
        out_shape=jax.ShapeDtypeStruct((_NP, 1), jnp.float32),
    )(q0, q1, z2, dis, b, s, t, h1, x, w1a, w1b, bias1, w2t, bias2,
      wab, wc, linb)


_MESH = plsc.VectorSubcoreMesh(core_axis_name="c", subcore_axis_name="s")


@functools.partial(
    pl.kernel,
    mesh=_MESH,
    out_type=jax.ShapeDtypeStruct((2 * _NP,), jnp.float32),
    scratch_types=[
        pltpu.VMEM((_CH,), jnp.int32),
        pltpu.VMEM((_CH,), jnp.int32),
        pltpu.VMEM((_CH,), jnp.float32),
        pltpu.VMEM((_CH,), jnp.float32),
        pltpu.VMEM((_SLICE,), jnp.float32),
        pltpu.VMEM_SHARED((_NP,), jnp.float32),
        pltpu.SemaphoreType.DMA,
        pltpu.SemaphoreType.DMA,
    ],
)
def _sc_deg(col_hbm, w_hbm, out_hbm, cidx0, cidx1, wch0, wch1, dbuf, acc,
            ds0, ds1):
    cid = lax.axis_index("c")
    sid = lax.axis_index("s")
    wid = cid * 16 + sid
    base = wid * _EPT

    def zero16(i, carry):
        dbuf[pl.ds(i * 16, 16)] = jnp.zeros((16,), jnp.float32)
        return carry

    lax.fori_loop(0, _SLICE // 16, zero16, 0)
    pltpu.sync_copy(dbuf, acc.at[pl.ds(sid * _SLICE, _SLICE)])
    plsc.subcore_barrier()

    pltpu.async_copy(col_hbm.at[pl.ds(base, _CH)], cidx0, ds0)
    pltpu.async_copy(w_hbm.at[pl.ds(base, _CH)], wch0, ds0)

    def pair(p, carry):
        for b, cidx, wch, sem, ocidx, owch, osem in (
                (0, cidx0, wch0, ds0, cidx1, wch1, ds1),
                (1, cidx1, wch1, ds1, cidx0, wch0, ds0)):
            k = 2 * p + b
            pltpu.make_async_copy(
                col_hbm.at[pl.ds(base + k * _CH, _CH)], cidx, sem).wait()
            pltpu.make_async_copy(
                w_hbm.at[pl.ds(base + k * _CH, _CH)], wch, sem).wait()

            @pl.when(k + 1 < _NCH)
            def _():
                pltpu.async_copy(
                    col_hbm.at[pl.ds(base + (k + 1) * _CH, _CH)], ocidx, osem)
                pltpu.async_copy(
                    w_hbm.at[pl.ds(base + (k + 1) * _CH, _CH)], owch, osem)

            pltpu.sync_copy(wch, acc.at[cidx], add=True)
        return carry

    lax.fori_loop(0, _NCH // 2, pair, 0)
    plsc.subcore_barrier()
    pltpu.sync_copy(acc.at[pl.ds(sid * _SLICE, _SLICE)], dbuf)
    pltpu.sync_copy(dbuf, out_hbm.at[pl.ds(cid * _NP + sid * _SLICE, _SLICE)])


@functools.partial(
    pl.kernel,
    mesh=_MESH,
    out_type=jax.ShapeDtypeStruct((2, _NP, _D), jnp.float32),
    scratch_types=[
        pltpu.VMEM((_CH,), jnp.int32),
        pltpu.VMEM((_CH,), jnp.int32),
        pltpu.VMEM((_CH,), jnp.int32),
        pltpu.VMEM((_CH,), jnp.int32),
        pltpu.VMEM((_CH,), jnp.float32),
        pltpu.VMEM((_CH,), jnp.float32),
        pltpu.VMEM((_CH, _D), jnp.float32),
        pltpu.VMEM((_CH, _D), jnp.float32),
        pltpu.VMEM_SHARED((_NP, _D), jnp.float32),
        pltpu.SemaphoreType.DMA,
        pltpu.SemaphoreType.DMA,
        pltpu.SemaphoreType.DMA,
        pltpu.SemaphoreType.DMA,
    ],
)
def _sc_agg(z_hbm, row_hbm, col_hbm, w_hbm, out_hbm,
            ridx0, ridx1, cidx0, cidx1, wch0, wch1, rows0, rows1, acc,
            is0, is1, gs0, gs1):
    cid = lax.axis_index("c")
    sid = lax.axis_index("s")
    ncht = jnp.where(cid == 0, _NC0, _NCH2 - _NC0)
    base = (jnp.where(cid == 0, sid * _NC0, 16 * _NC0 + sid * (_NCH2 - _NC0))
            * _CH)

    def idx_issue(off, ridx, cidx, wch, isem):
        pltpu.async_copy(row_hbm.at[pl.ds(off, _CH)], ridx, isem)
        pltpu.async_copy(col_hbm.at[pl.ds(off, _CH)], cidx, isem)
        pltpu.async_copy(w_hbm.at[pl.ds(off, _CH)], wch, isem)

    def idx_wait(off, ridx, cidx, wch, isem):
        pltpu.make_async_copy(row_hbm.at[pl.ds(off, _CH)], ridx, isem).wait()
        pltpu.make_async_copy(col_hbm.at[pl.ds(off, _CH)], cidx, isem).wait()
        pltpu.make_async_copy(w_hbm.at[pl.ds(off, _CH)], wch, isem).wait()

    idx_issue(base, ridx0, cidx0, wch0, is0)

    def zrow(r, carry):
        for j in range(8):
            rows1[r, pl.ds(j * 16, 16)] = jnp.zeros((16,), jnp.float32)
        return carry

    lax.fori_loop(0, _CH, zrow, 0)
    for p in range(_SLICE // _CH):
        pltpu.sync_copy(rows1, acc.at[pl.ds(sid * _SLICE + p * _CH, _CH)])

    idx_wait(base, ridx0, cidx0, wch0, is0)
    pltpu.async_copy(z_hbm.at[ridx0], rows0, gs0)
    plsc.subcore_barrier()

    def pair(p, carry):
        for b, ridx, cidx, wch, isem, rows, gs, \
                oridx, ocidx, owch, oisem, orows, ogs in (
                (0, ridx0, cidx0, wch0, is0, rows0, gs0,
                 ridx1, cidx1, wch1, is1, rows1, gs1),
                (1, ridx1, cidx1, wch1, is1, rows1, gs1,
                 ridx0, cidx0, wch0, is0, rows0, gs0)):
            k = 2 * p + b
            nxt_off = base + (k + 1) * _CH

            @pl.when(k + 1 < ncht)
            def _():
                idx_issue(nxt_off, oridx, ocidx, owch, oisem)

            pltpu.make_async_copy(z_hbm.at[ridx], rows, gs).wait()

            def scale(g, c2):
                wv16 = wch[pl.ds(g * 16, 16)]
                for l in range(16):
                    e = g * 16 + l
                    wv = jnp.full((16,), wv16[l])
                    for j in range(8):
                        sl = pl.ds(j * 16, 16)
                        rows[e, sl] = rows[e, sl] * wv
                return c2

            lax.fori_loop(0, _CH // 16, scale, 0)

            @pl.when(k + 1 < ncht)
            def _():
                idx_wait(nxt_off, oridx, ocidx, owch, oisem)
                pltpu.async_copy(z_hbm.at[oridx], orows, ogs)

            pltpu.sync_copy(rows, acc.at[cidx], add=True)
        return carry

    lax.fori_loop(0, ncht // 2, pair, 0)
    plsc.subcore_barrier()
    np_ = _SLICE // _CH
    bufs = (rows0, rows1)
    sems = (gs0, gs1)
    rd = pltpu.async_copy(acc.at[pl.ds(sid * _SLICE, _CH)], rows0, gs0)
    for p in range(np_):
        off = sid * _SLICE + p * _CH
        rd.wait()
        if p + 1 < np_:
            rd = pltpu.async_copy(
                acc.at[pl.ds(off + _CH, _CH)], bufs[(p + 1) % 2],
                sems[(p + 1) % 2])
        pltpu.sync_copy(bufs[p % 2], out_hbm.at[cid, pl.ds(off, _CH)])


def kernel(x, edge_index, edge_weight, W1, b1, bn1_g, bn1_b, bn1_rm, bn1_rv,
           W2, b2, bn2_g, bn2_b, bn2_rm, bn2_rv,
           l1_wih, l1_whh, l1_bih, l1_bhh, l2_wih, l2_whh, l2_bih, l2_bhh,
           lin_w, lin_b):
    f32 = jnp.float32
    row = edge_index[0]
    col = edge_index[1]
    xp = jnp.zeros((_NP, _D), f32).at[:_N].set(x)
    rowp = jnp.zeros((_EP,), jnp.int32).at[:_E].set(row)
    colp = jnp.zeros((_EP,), jnp.int32).at[:_E].set(col)
    wp = jnp.zeros((_EP,), f32).at[:_E].set(edge_weight)

    s1 = (bn1_g / jnp.sqrt(bn1_rv + 1e-5)).reshape(1, _D)
    t1 = (bn1_b - bn1_rm * s1[0]).reshape(1, _D)
    s2 = (bn2_g / jnp.sqrt(bn2_rv + 1e-5)).reshape(1, _D)
    t2 = (bn2_b - bn2_rm * s2[0]).reshape(1, _D)

    w1t = l1_wih.T
    w1a = w1t[:_D]
    w1b = w1t[_D:]
    bias1 = (l1_bih + l1_bhh).reshape(1, 4 * _D)
    w2t = l2_wih.T
    bias2 = (l2_bih + l2_bhh).reshape(1, 4 * _D)
    wab = (lin_w[0, :_D] + lin_w[0, _D:2 * _D]).reshape(_D, 1)
    wc = lin_w[0, 2 * _D:].reshape(_D, 1)
    linb = lin_b.reshape(1, 1)

    deg_p = _sc_deg(colp, wp)
    dis, z1 = _tc_a(deg_p[:_NP].reshape(_NP, 1), deg_p[_NP:].reshape(_NP, 1),
                    xp, W1)

    q1 = _sc_agg(z1, rowp, colp, wp)
    h1, z2 = _tc_b(q1[0], q1[1], z1, dis, b1.reshape(1, _D), s1, t1, W2)

    q2 = _sc_agg(z2, rowp, colp, wp)
    out = _tc_c(q2[0], q2[1], z2, dis, b2.reshape(1, _D), s2, t2, h1, xp,
                w1a, w1b, bias1, w2t, bias2, wab, wc, linb)
    return out[:_N]
